# Initial kernel scaffold; baseline (speedup 1.0000x reference)
import jax
import jax.numpy as jnp
from jax.experimental import pallas as pl


def kernel(input, input_timestamp, edge_index, emb, W1, b1, W2, b2):
    raise NotImplementedError("write your pallas kernel here")


# speedup vs baseline: 24.7153x; 24.7153x over previous
import functools

import jax
import jax.numpy as jnp
from jax import lax
from jax.experimental import pallas as pl
from jax.experimental.pallas import tpu as pltpu
from jax.experimental.pallas import tpu_sc as plsc

N = 50000
F = 32
FH = F // 2
NP = 50176
E = 800000
EPAD = 819200
EBLK = EPAD // 128
PAD_DST = NP - 8
NC, NS = 2, 16
NP_S = NP // NS
WBLK = EBLK // NS

_MESH = plsc.VectorSubcoreMesh(
    core_axis_name="c", subcore_axis_name="s", num_cores=NC, num_subcores=NS)
_SC_PARAMS = pltpu.CompilerParams(
    needs_layout_passes=False, use_tc_tiling_on_sc=False)

_Z16 = functools.partial(jnp.zeros, (16,), jnp.float32)


def _deg_body(dst_hbm, deg_hbm, hist, dbuf):
    c = lax.axis_index("c")
    s = lax.axis_index("s")
    w = c * NS + s
    ones = jnp.ones((16,), jnp.float32)

    def zero_body(j, _):
        hist[pl.ds(j * 16, 16)] = _Z16()
        return 0
    lax.fori_loop(0, NP // 16, zero_body, 0)

    ew = EPAD // (NC * NS)
    kb = 3200

    def outer(b, _):
        pltpu.sync_copy(dst_hbm.at[pl.ds(w * ew + b * kb, kb)], dbuf)

        def inner(j, _):
            iv = dbuf[pl.ds(j * 16, 16)]
            plsc.addupdate_scatter(hist, [iv], ones)
            return 0
        lax.fori_loop(0, kb // 16, inner, 0)
        return 0
    lax.fori_loop(0, ew // kb, outer, 0)

    pltpu.sync_copy(hist, deg_hbm.at[pl.ds(w * NP, NP)])


def _deg_call(dst_flat):
    return pl.kernel(
        _deg_body,
        out_type=jax.ShapeDtypeStruct((NC * NS * NP,), jnp.float32),
        mesh=_MESH,
        compiler_params=_SC_PARAMS,
        scratch_types=[
            pltpu.VMEM((NP,), jnp.float32),
            pltpu.VMEM((3200,), jnp.int32),
        ],
    )(dst_flat)


_KI = 16
_KW = 8
_ZCH = 1568


def _scatter_body(x_hbm, src_hbm, dst_hbm, out_hbm,
                  srcv, dstv, rowsa, rowsb, acc, semg, sems):
    c = lax.axis_index("c")
    s = lax.axis_index("s")

    def zrow(j, _):
        rowsa[j, pl.ds(0, 16)] = _Z16()
        return 0
    lax.fori_loop(0, _ZCH, zrow, 0)
    for q in range(NP_S // _ZCH):
        pltpu.sync_copy(rowsa.at[pl.ds(0, _ZCH), :],
                        acc.at[pl.ds(s * NP_S + q * _ZCH, _ZCH), :])
    plsc.subcore_barrier()

    def outer(b, _):
        blk0 = s * WBLK + b * _KI
        pltpu.sync_copy(src_hbm.at[c, pl.ds(blk0, _KI), :], srcv)
        pltpu.sync_copy(dst_hbm.at[pl.ds(blk0, _KI), :], dstv)
        ga = [pltpu.async_copy(x_hbm.at[srcv.at[j]],
                               rowsa.at[pl.ds(j * 128, 128), :], semg)
              for j in range(_KW)]
        gb = [pltpu.async_copy(x_hbm.at[srcv.at[_KW + j]],
                               rowsb.at[pl.ds(j * 128, 128), :], semg)
              for j in range(_KW)]
        for d in ga:
            d.wait()
        sa = [pltpu.async_copy(rowsa.at[pl.ds(j * 128, 128), :],
                               acc.at[dstv.at[j]], sems, add=True)
              for j in range(_KW)]
        for d in gb:
            d.wait()
        sb = [pltpu.async_copy(rowsb.at[pl.ds(j * 128, 128), :],
                               acc.at[dstv.at[_KW + j]], sems, add=True)
              for j in range(_KW)]
        for d in sa:
            d.wait()
        for d in sb:
            d.wait()
        return 0
    lax.fori_loop(0, WBLK // _KI, outer, 0)
    plsc.subcore_barrier()

    for q in range(NP_S // _ZCH):
        pltpu.sync_copy(acc.at[pl.ds(s * NP_S + q * _ZCH, _ZCH), :],
                        rowsa.at[pl.ds(0, _ZCH), :])
        pltpu.sync_copy(rowsa.at[pl.ds(0, _ZCH), :],
                        out_hbm.at[c, pl.ds(s * NP_S + q * _ZCH, _ZCH), :])


def _scatter_call(xcat, srcall, dst3):
    return pl.kernel(
        _scatter_body,
        out_type=jax.ShapeDtypeStruct((NC, NP, FH), jnp.float32),
        mesh=_MESH,
        compiler_params=_SC_PARAMS,
        scratch_types=[
            pltpu.VMEM((_KI, 128), jnp.int32),
            pltpu.VMEM((_KI, 128), jnp.int32),
            pltpu.VMEM((_KW * 128, FH), jnp.float32),
            pltpu.VMEM((_KW * 128, FH), jnp.float32),
            pltpu.VMEM_SHARED((NP, FH), jnp.float32),
            pltpu.SemaphoreType.DMA,
            pltpu.SemaphoreType.DMA,
        ],
    )(xcat, srcall, dst3)


_GK = 10
_GOUT = 204800
_GBLK = _GOUT // 128 // (NC * NS)


def _gather_body(tbl_hbm, idx_hbm, out_hbm, idxv, rows, sem):
    c = lax.axis_index("c")
    s = lax.axis_index("s")
    w = c * NS + s
    pltpu.sync_copy(idx_hbm.at[w], idxv)

    def outer(b, _):
        t0 = b * _GK
        descs = [
            pltpu.async_copy(tbl_hbm.at[idxv.at[t0 + j]],
                             rows.at[pl.ds(j * 128, 128), :], sem)
            for j in range(_GK)
        ]
        for d in descs:
            d.wait()
        pltpu.sync_copy(
            rows, out_hbm.at[pl.ds((w * _GBLK + t0) * 128, _GK * 128), :])
        return 0
    lax.fori_loop(0, _GBLK // _GK, outer, 0)


def _gather_call(table, idx3):
    return pl.kernel(
        _gather_body,
        out_type=jax.ShapeDtypeStruct((_GOUT, F), jnp.float32),
        mesh=_MESH,
        compiler_params=_SC_PARAMS,
        scratch_types=[
            pltpu.VMEM((_GBLK, 128), jnp.int32),
            pltpu.VMEM((_GK * 128, F), jnp.float32),
            pltpu.SemaphoreType.DMA,
        ],
    )(table, idx3)


_R = NP // 8
_GRID = NP // _R


def _row_spec():
    return pl.BlockSpec((_R, F), lambda i: (i, 0))


def _vec_spec():
    return pl.BlockSpec((_R, 1), lambda i: (i, 0))


def _half_spec():
    return pl.BlockSpec((NC, _R, FH), lambda i: (0, i, 0))


def _full_spec(shape):
    nd = len(shape)
    return pl.BlockSpec(shape, lambda i: (0,) * nd)


def _prep_body(deg_ref, emb_ref, dinv_ref, x0_ref):
    deg = jnp.sum(deg_ref[...], axis=0) + 1.0
    dinv = lax.rsqrt(deg)[:, None]
    dinv_ref[...] = dinv
    x0 = emb_ref[...] * dinv
    x0_ref[0] = x0[:, :FH]
    x0_ref[1] = x0[:, FH:]


def _prep_call(degp, emb_p):
    return pl.pallas_call(
        _prep_body,
        grid=(_GRID,),
        in_specs=[pl.BlockSpec((NC * NS, _R), lambda i: (0, i)),
                  _row_spec()],
        out_specs=(_vec_spec(), _half_spec()),
        out_shape=(jax.ShapeDtypeStruct((NP, 1), jnp.float32),
                   jax.ShapeDtypeStruct((NC, NP, FH), jnp.float32)),
    )(degp, emb_p)


def _mid_body(scat_ref, emb_ref, dinv_ref, w1_ref, b1_ref, w2_ref, g_ref):
    dinv = dinv_ref[...]
    s1 = (emb_ref[...] * dinv
          + jnp.concatenate([scat_ref[0], scat_ref[1]], axis=1))
    h = jnp.dot(s1 * dinv, w1_ref[...],
                preferred_element_type=jnp.float32) + b1_ref[...][None, :]
    g = jnp.dot(h, w2_ref[...], preferred_element_type=jnp.float32) * dinv
    g_ref[0] = g[:, :FH]
    g_ref[1] = g[:, FH:]


def _mid_call(scat1, emb_p, dinv, W1, b1, W2):
    return pl.pallas_call(
        _mid_body,
        grid=(_GRID,),
        in_specs=[_half_spec(), _row_spec(), _vec_spec(),
                  _full_spec((F, 2 * F)), _full_spec((2 * F,)),
                  _full_spec((2 * F, F))],
        out_specs=_half_spec(),
        out_shape=jax.ShapeDtypeStruct((NC, NP, FH), jnp.float32),
    )(scat1, emb_p, dinv, W1, b1, W2)


def _fin_body(scat_ref, g_ref, dinv_ref, b2_ref, tbl_ref):
    gv = jnp.concatenate([g_ref[0], g_ref[1]], axis=1)
    sv = jnp.concatenate([scat_ref[0], scat_ref[1]], axis=1)
    tbl_ref[...] = (gv + sv) * dinv_ref[...] + b2_ref[...][None, :]


def _fin_call(scat2, gcat, dinv, b2):
    return pl.pallas_call(
        _fin_body,
        grid=(_GRID,),
        in_specs=[_half_spec(), _half_spec(), _vec_spec(),
                  _full_spec((F,))],
        out_specs=_row_spec(),
        out_shape=jax.ShapeDtypeStruct((NP, F), jnp.float32),
    )(scat2, gcat, dinv, b2)


def kernel(input, input_timestamp, edge_index, emb, W1, b1, W2, b2):
    del input_timestamp
    src = edge_index[0].astype(jnp.int32)
    dst = edge_index[1].astype(jnp.int32)
    npad = EPAD - E
    srcp = jnp.concatenate([src, jnp.zeros((npad,), jnp.int32)])
    dstp = jnp.concatenate([dst, jnp.full((npad,), PAD_DST, jnp.int32)])
    srcall = jnp.stack([srcp, srcp + NP]).reshape(NC, EBLK, 128)
    dst3 = dstp.reshape(EBLK, 128)

    emb_p = jnp.zeros((NP, F), jnp.float32).at[:N].set(emb)

    degp = _deg_call(dstp).reshape(NC * NS, NP)
    dinv, x0cat = _prep_call(degp, emb_p)
    scat1 = _scatter_call(x0cat.reshape(NC * NP, FH), srcall, dst3)
    gcat = _mid_call(scat1, emb_p, dinv, W1, b1, W2)
    scat2 = _scatter_call(gcat.reshape(NC * NP, FH), srcall, dst3)
    table = _fin_call(scat2, gcat, dinv, b2)

    flat = input.reshape(-1).astype(jnp.int32)
    out = _gather_call(table, flat.reshape(NC * NS, _GBLK, 128))
    return out.reshape(input.shape[0], input.shape[1], F)


# speedup vs baseline: 25.3710x; 1.0265x over previous
import functools

import jax
import jax.numpy as jnp
from jax import lax
from jax.experimental import pallas as pl
from jax.experimental.pallas import tpu as pltpu
from jax.experimental.pallas import tpu_sc as plsc

N = 50000
F = 32
FH = F // 2
NP = 50176
E = 800000
EPAD = 819200
EBLK = EPAD // 128
PAD_DST = NP - 8
NC, NS = 2, 16
NP_S = NP // NS
WBLK = EBLK // NS

_MESH = plsc.VectorSubcoreMesh(
    core_axis_name="c", subcore_axis_name="s", num_cores=NC, num_subcores=NS)
_SC_PARAMS = pltpu.CompilerParams(
    needs_layout_passes=False, use_tc_tiling_on_sc=False)

_Z16 = functools.partial(jnp.zeros, (16,), jnp.float32)


def _deg_body(dst_hbm, deg_hbm, hist, dbuf):
    c = lax.axis_index("c")
    s = lax.axis_index("s")
    w = c * NS + s
    ones = jnp.ones((16,), jnp.float32)

    def zero_body(j, _):
        hist[pl.ds(j * 16, 16)] = _Z16()
        return 0
    lax.fori_loop(0, NP // 16, zero_body, 0)

    ew = EPAD // (NC * NS)
    kb = 3200

    def outer(b, _):
        pltpu.sync_copy(dst_hbm.at[pl.ds(w * ew + b * kb, kb)], dbuf)

        def inner(j, _):
            iv = dbuf[pl.ds(j * 16, 16)]
            plsc.addupdate_scatter(hist, [iv], ones)
            return 0
        lax.fori_loop(0, kb // 16, inner, 0)
        return 0
    lax.fori_loop(0, ew // kb, outer, 0)

    pltpu.sync_copy(hist, deg_hbm.at[pl.ds(w * NP, NP)])


def _deg_call(dst_flat):
    return pl.kernel(
        _deg_body,
        out_type=jax.ShapeDtypeStruct((NC * NS * NP,), jnp.float32),
        mesh=_MESH,
        compiler_params=_SC_PARAMS,
        scratch_types=[
            pltpu.VMEM((NP,), jnp.float32),
            pltpu.VMEM((3200,), jnp.int32),
        ],
    )(dst_flat)


_KI = 16
_KW = 8
_ZCH = 1568


def _scatter_body(x_hbm, src_hbm, dst_hbm, out_hbm,
                  srcv, dstv, rowsa, rowsb, acc, semg, sems):
    c = lax.axis_index("c")
    s = lax.axis_index("s")

    def zrow(j, _):
        rowsa[j, pl.ds(0, 16)] = _Z16()
        return 0
    lax.fori_loop(0, _ZCH, zrow, 0)
    for q in range(NP_S // _ZCH):
        pltpu.sync_copy(rowsa.at[pl.ds(0, _ZCH), :],
                        acc.at[pl.ds(s * NP_S + q * _ZCH, _ZCH), :])
    plsc.subcore_barrier()

    xc = x_hbm.at[c]
    def outer(b, _):
        blk0 = s * WBLK + b * _KI
        pltpu.sync_copy(src_hbm.at[pl.ds(blk0, _KI), :], srcv)
        pltpu.sync_copy(dst_hbm.at[pl.ds(blk0, _KI), :], dstv)
        ga = [pltpu.async_copy(xc.at[srcv.at[j]],
                               rowsa.at[pl.ds(j * 128, 128), :], semg)
              for j in range(_KW)]
        gb = [pltpu.async_copy(xc.at[srcv.at[_KW + j]],
                               rowsb.at[pl.ds(j * 128, 128), :], semg)
              for j in range(_KW)]
        for d in ga:
            d.wait()
        sa = [pltpu.async_copy(rowsa.at[pl.ds(j * 128, 128), :],
                               acc.at[dstv.at[j]], sems, add=True)
              for j in range(_KW)]
        for d in gb:
            d.wait()
        sb = [pltpu.async_copy(rowsb.at[pl.ds(j * 128, 128), :],
                               acc.at[dstv.at[_KW + j]], sems, add=True)
              for j in range(_KW)]
        for d in sa:
            d.wait()
        for d in sb:
            d.wait()
        return 0
    lax.fori_loop(0, WBLK // _KI, outer, 0)
    plsc.subcore_barrier()

    for q in range(NP_S // _ZCH):
        pltpu.sync_copy(acc.at[pl.ds(s * NP_S + q * _ZCH, _ZCH), :],
                        rowsa.at[pl.ds(0, _ZCH), :])
        pltpu.sync_copy(rowsa.at[pl.ds(0, _ZCH), :],
                        out_hbm.at[c, pl.ds(s * NP_S + q * _ZCH, _ZCH), :])


def _scatter_call(xcat, src3, dst3):
    return pl.kernel(
        _scatter_body,
        out_type=jax.ShapeDtypeStruct((NC, NP, FH), jnp.float32),
        mesh=_MESH,
        compiler_params=_SC_PARAMS,
        scratch_types=[
            pltpu.VMEM((_KI, 128), jnp.int32),
            pltpu.VMEM((_KI, 128), jnp.int32),
            pltpu.VMEM((_KW * 128, FH), jnp.float32),
            pltpu.VMEM((_KW * 128, FH), jnp.float32),
            pltpu.VMEM_SHARED((NP, FH), jnp.float32),
            pltpu.SemaphoreType.DMA,
            pltpu.SemaphoreType.DMA,
        ],
    )(xcat, src3, dst3)


_GK = 10
_GOUT = 204800
_GBLK = _GOUT // 128 // (NC * NS)


def _gather_body(tbl_hbm, idx_hbm, out_hbm, idxv, rows, sem):
    c = lax.axis_index("c")
    s = lax.axis_index("s")
    w = c * NS + s
    pltpu.sync_copy(idx_hbm.at[w], idxv)

    def outer(b, _):
        t0 = b * _GK
        descs = [
            pltpu.async_copy(tbl_hbm.at[idxv.at[t0 + j]],
                             rows.at[pl.ds(j * 128, 128), :], sem)
            for j in range(_GK)
        ]
        for d in descs:
            d.wait()
        pltpu.sync_copy(
            rows, out_hbm.at[pl.ds((w * _GBLK + t0) * 128, _GK * 128), :])
        return 0
    lax.fori_loop(0, _GBLK // _GK, outer, 0)


def _gather_call(table, idx3):
    return pl.kernel(
        _gather_body,
        out_type=jax.ShapeDtypeStruct((_GOUT, F), jnp.float32),
        mesh=_MESH,
        compiler_params=_SC_PARAMS,
        scratch_types=[
            pltpu.VMEM((_GBLK, 128), jnp.int32),
            pltpu.VMEM((_GK * 128, F), jnp.float32),
            pltpu.SemaphoreType.DMA,
        ],
    )(table, idx3)


_R = NP // 8
_GRID = NP // _R


def _row_spec():
    return pl.BlockSpec((_R, F), lambda i: (i, 0))


def _vec_spec():
    return pl.BlockSpec((_R, 1), lambda i: (i, 0))


def _half_spec():
    return pl.BlockSpec((NC, _R, FH), lambda i: (0, i, 0))


def _full_spec(shape):
    nd = len(shape)
    return pl.BlockSpec(shape, lambda i: (0,) * nd)


def _prep_body(deg_ref, emb_ref, dinv_ref, x0_ref):
    deg = jnp.sum(deg_ref[...], axis=0) + 1.0
    dinv = lax.rsqrt(deg)[:, None]
    dinv_ref[...] = dinv
    x0 = emb_ref[...] * dinv
    x0_ref[0] = x0[:, :FH]
    x0_ref[1] = x0[:, FH:]


def _prep_call(degp, emb_p):
    return pl.pallas_call(
        _prep_body,
        grid=(_GRID,),
        in_specs=[pl.BlockSpec((NC * NS, _R), lambda i: (0, i)),
                  _row_spec()],
        out_specs=(_vec_spec(), _half_spec()),
        out_shape=(jax.ShapeDtypeStruct((NP, 1), jnp.float32),
                   jax.ShapeDtypeStruct((NC, NP, FH), jnp.float32)),
    )(degp, emb_p)


def _mid_body(scat_ref, emb_ref, dinv_ref, w1_ref, b1_ref, w2_ref, g_ref):
    dinv = dinv_ref[...]
    s1 = (emb_ref[...] * dinv
          + jnp.concatenate([scat_ref[0], scat_ref[1]], axis=1))
    h = jnp.dot(s1 * dinv, w1_ref[...],
                preferred_element_type=jnp.float32) + b1_ref[...][None, :]
    g = jnp.dot(h, w2_ref[...], preferred_element_type=jnp.float32) * dinv
    g_ref[0] = g[:, :FH]
    g_ref[1] = g[:, FH:]


def _mid_call(scat1, emb_p, dinv, W1, b1, W2):
    return pl.pallas_call(
        _mid_body,
        grid=(_GRID,),
        in_specs=[_half_spec(), _row_spec(), _vec_spec(),
                  _full_spec((F, 2 * F)), _full_spec((2 * F,)),
                  _full_spec((2 * F, F))],
        out_specs=_half_spec(),
        out_shape=jax.ShapeDtypeStruct((NC, NP, FH), jnp.float32),
    )(scat1, emb_p, dinv, W1, b1, W2)


def _fin_body(scat_ref, g_ref, dinv_ref, b2_ref, tbl_ref):
    gv = jnp.concatenate([g_ref[0], g_ref[1]], axis=1)
    sv = jnp.concatenate([scat_ref[0], scat_ref[1]], axis=1)
    tbl_ref[...] = (gv + sv) * dinv_ref[...] + b2_ref[...][None, :]


def _fin_call(scat2, gcat, dinv, b2):
    return pl.pallas_call(
        _fin_body,
        grid=(_GRID,),
        in_specs=[_half_spec(), _half_spec(), _vec_spec(),
                  _full_spec((F,))],
        out_specs=_row_spec(),
        out_shape=jax.ShapeDtypeStruct((NP, F), jnp.float32),
    )(scat2, gcat, dinv, b2)


def kernel(input, input_timestamp, edge_index, emb, W1, b1, W2, b2):
    del input_timestamp
    src = edge_index[0].astype(jnp.int32)
    dst = edge_index[1].astype(jnp.int32)
    npad = EPAD - E
    srcp = jnp.concatenate([src, jnp.zeros((npad,), jnp.int32)])
    dstp = jnp.concatenate([dst, jnp.full((npad,), PAD_DST, jnp.int32)])
    src3 = srcp.reshape(EBLK, 128)
    dst3 = dstp.reshape(EBLK, 128)

    emb_p = jnp.zeros((NP, F), jnp.float32).at[:N].set(emb)

    degp = _deg_call(dstp).reshape(NC * NS, NP)
    dinv, x0cat = _prep_call(degp, emb_p)
    scat1 = _scatter_call(x0cat, src3, dst3)
    gcat = _mid_call(scat1, emb_p, dinv, W1, b1, W2)
    scat2 = _scatter_call(gcat, src3, dst3)
    table = _fin_call(scat2, gcat, dinv, b2)

    flat = input.reshape(-1).astype(jnp.int32)
    out = _gather_call(table, flat.reshape(NC * NS, _GBLK, 128))
    return out.reshape(input.shape[0], input.shape[1], F)


# speedup vs baseline: 26.1339x; 1.0301x over previous
import functools

import jax
import jax.numpy as jnp
from jax import lax
from jax.experimental import pallas as pl
from jax.experimental.pallas import tpu as pltpu
from jax.experimental.pallas import tpu_sc as plsc

N = 50000
F = 32
FH = F // 2
NP = 50176
E = 800000
EPAD = 819200
EBLK = EPAD // 128
PAD_DST = NP - 8
NC, NS = 2, 16
NP_S = NP // NS
WBLK = EBLK // NS

_MESH = plsc.VectorSubcoreMesh(
    core_axis_name="c", subcore_axis_name="s", num_cores=NC, num_subcores=NS)
_SC_PARAMS = pltpu.CompilerParams(
    needs_layout_passes=False, use_tc_tiling_on_sc=False)

_Z16 = functools.partial(jnp.zeros, (16,), jnp.float32)


def _deg_body(dst_hbm, deg_hbm, hist, dbuf):
    c = lax.axis_index("c")
    s = lax.axis_index("s")
    w = c * NS + s
    ones = jnp.ones((16,), jnp.float32)

    def zero_body(j, _):
        hist[pl.ds(j * 16, 16)] = _Z16()
        return 0
    lax.fori_loop(0, NP // 16, zero_body, 0)

    ew = EPAD // (NC * NS)
    kb = 3200

    def outer(b, _):
        pltpu.sync_copy(dst_hbm.at[pl.ds(w * ew + b * kb, kb)], dbuf)

        def inner(j, _):
            iv = dbuf[pl.ds(j * 16, 16)]
            plsc.addupdate_scatter(hist, [iv], ones)
            return 0
        lax.fori_loop(0, kb // 16, inner, 0)
        return 0
    lax.fori_loop(0, ew // kb, outer, 0)

    pltpu.sync_copy(hist, deg_hbm.at[pl.ds(w * NP, NP)])


def _deg_call(dst_flat):
    return pl.kernel(
        _deg_body,
        out_type=jax.ShapeDtypeStruct((NC * NS * NP,), jnp.float32),
        mesh=_MESH,
        compiler_params=_SC_PARAMS,
        scratch_types=[
            pltpu.VMEM((NP,), jnp.float32),
            pltpu.VMEM((3200,), jnp.int32),
        ],
    )(dst_flat)


_KI = 16
_KW = 8
_ZCH = 1568


def _scatter_body(x_hbm, src_hbm, dst_hbm, out_hbm,
                  srcv, dstv, rowsa, rowsb, acc, semg, sems):
    c = lax.axis_index("c")
    s = lax.axis_index("s")

    def zrow(j, _):
        rowsa[j, pl.ds(0, 16)] = _Z16()
        return 0
    lax.fori_loop(0, _ZCH, zrow, 0)
    for q in range(NP_S // _ZCH):
        pltpu.sync_copy(rowsa.at[pl.ds(0, _ZCH), :],
                        acc.at[pl.ds(s * NP_S + q * _ZCH, _ZCH), :])
    plsc.subcore_barrier()

    xc = x_hbm.at[c]
    def outer(b, _):
        blk0 = s * WBLK + b * _KI
        pltpu.sync_copy(src_hbm.at[pl.ds(blk0, _KI), :], srcv)
        pltpu.sync_copy(dst_hbm.at[pl.ds(blk0, _KI), :], dstv)
        ga = [pltpu.async_copy(xc.at[srcv.at[j]],
                               rowsa.at[pl.ds(j * 128, 128), :], semg)
              for j in range(_KW)]
        gb = [pltpu.async_copy(xc.at[srcv.at[_KW + j]],
                               rowsb.at[pl.ds(j * 128, 128), :], semg)
              for j in range(_KW)]
        for d in ga:
            d.wait()
        sa = [pltpu.async_copy(rowsa.at[pl.ds(j * 128, 128), :],
                               acc.at[dstv.at[j]], sems, add=True)
              for j in range(_KW)]
        for d in gb:
            d.wait()
        sb = [pltpu.async_copy(rowsb.at[pl.ds(j * 128, 128), :],
                               acc.at[dstv.at[_KW + j]], sems, add=True)
              for j in range(_KW)]
        for d in sa:
            d.wait()
        for d in sb:
            d.wait()
        return 0
    lax.fori_loop(0, WBLK // _KI, outer, 0)
    plsc.subcore_barrier()

    for q in range(NP_S // _ZCH):
        pltpu.sync_copy(acc.at[pl.ds(s * NP_S + q * _ZCH, _ZCH), :],
                        rowsa.at[pl.ds(0, _ZCH), :])
        pltpu.sync_copy(rowsa.at[pl.ds(0, _ZCH), :],
                        out_hbm.at[c, pl.ds(s * NP_S + q * _ZCH, _ZCH), :])


def _scatter_call(xcat, src3, dst3):
    return pl.kernel(
        _scatter_body,
        out_type=jax.ShapeDtypeStruct((NC, NP, FH), jnp.float32),
        mesh=_MESH,
        compiler_params=_SC_PARAMS,
        scratch_types=[
            pltpu.VMEM((_KI, 128), jnp.int32),
            pltpu.VMEM((_KI, 128), jnp.int32),
            pltpu.VMEM((_KW * 128, FH), jnp.float32),
            pltpu.VMEM((_KW * 128, FH), jnp.float32),
            pltpu.VMEM_SHARED((NP, FH), jnp.float32),
            pltpu.SemaphoreType.DMA,
            pltpu.SemaphoreType.DMA,
        ],
    )(xcat, src3, dst3)


_GK = 10
_GOUT = 204800
_GBLK = _GOUT // 128 // (NC * NS)


def _gather_body(tbl_hbm, idx_hbm, out_hbm, idxv, rows, sem):
    c = lax.axis_index("c")
    s = lax.axis_index("s")
    w = c * NS + s
    pltpu.sync_copy(idx_hbm.at[w], idxv)

    def outer(b, _):
        t0 = b * _GK
        descs = [
            pltpu.async_copy(tbl_hbm.at[idxv.at[t0 + j]],
                             rows.at[pl.ds(j * 128, 128), :], sem)
            for j in range(_GK)
        ]
        for d in descs:
            d.wait()
        pltpu.sync_copy(
            rows, out_hbm.at[pl.ds((w * _GBLK + t0) * 128, _GK * 128), :])
        return 0
    lax.fori_loop(0, _GBLK // _GK, outer, 0)


def _gather_call(table, idx3):
    return pl.kernel(
        _gather_body,
        out_type=jax.ShapeDtypeStruct((_GOUT, F), jnp.float32),
        mesh=_MESH,
        compiler_params=_SC_PARAMS,
        scratch_types=[
            pltpu.VMEM((_GBLK, 128), jnp.int32),
            pltpu.VMEM((_GK * 128, F), jnp.float32),
            pltpu.SemaphoreType.DMA,
        ],
    )(table, idx3)


_R = NP // 8
_GRID = NP // _R


def _row_spec():
    return pl.BlockSpec((_R, F), lambda i: (i, 0))


def _vec_spec():
    return pl.BlockSpec((_R, 1), lambda i: (i, 0))


def _half_spec():
    return pl.BlockSpec((NC, _R, FH), lambda i: (0, i, 0))


def _full_spec(shape):
    nd = len(shape)
    return pl.BlockSpec(shape, lambda i: (0,) * nd)


def _prep_body(deg_ref, emb_ref, dinv_ref, x0_ref):
    deg = jnp.sum(deg_ref[...], axis=0) + 1.0
    dinv = lax.rsqrt(deg)[:, None]
    dinv_ref[...] = dinv
    x0 = emb_ref[...] * dinv
    x0_ref[0] = x0[:, :FH]
    x0_ref[1] = x0[:, FH:]


def _prep_call(degp, emb_p):
    return pl.pallas_call(
        _prep_body,
        grid=(_GRID,),
        in_specs=[pl.BlockSpec((NC * NS, _R), lambda i: (0, i)),
                  _row_spec()],
        out_specs=(_vec_spec(), _half_spec()),
        out_shape=(jax.ShapeDtypeStruct((NP, 1), jnp.float32),
                   jax.ShapeDtypeStruct((NC, NP, FH), jnp.float32)),
    )(degp, emb_p)


def _mid_body(scat_ref, emb_ref, dinv_ref, w1_ref, b1_ref, w2_ref, g_ref):
    dinv = dinv_ref[...]
    s1 = (emb_ref[...] * dinv
          + jnp.concatenate([scat_ref[0], scat_ref[1]], axis=1))
    h = jnp.dot(s1 * dinv, w1_ref[...],
                preferred_element_type=jnp.float32) + b1_ref[...][None, :]
    g = jnp.dot(h, w2_ref[...], preferred_element_type=jnp.float32) * dinv
    g_ref[0] = g[:, :FH]
    g_ref[1] = g[:, FH:]


def _mid_call(scat1, emb_p, dinv, W1, b1, W2):
    return pl.pallas_call(
        _mid_body,
        grid=(_GRID,),
        in_specs=[_half_spec(), _row_spec(), _vec_spec(),
                  _full_spec((F, 2 * F)), _full_spec((2 * F,)),
                  _full_spec((2 * F, F))],
        out_specs=_half_spec(),
        out_shape=jax.ShapeDtypeStruct((NC, NP, FH), jnp.float32),
    )(scat1, emb_p, dinv, W1, b1, W2)


def _fin_body(scat_ref, g_ref, dinv_ref, b2_ref, tbl_ref):
    gv = jnp.concatenate([g_ref[0], g_ref[1]], axis=1)
    sv = jnp.concatenate([scat_ref[0], scat_ref[1]], axis=1)
    tbl_ref[...] = (gv + sv) * dinv_ref[...] + b2_ref[...][None, :]


def _fin_call(scat2, gcat, dinv, b2):
    return pl.pallas_call(
        _fin_body,
        grid=(_GRID,),
        in_specs=[_half_spec(), _half_spec(), _vec_spec(),
                  _full_spec((F,))],
        out_specs=_row_spec(),
        out_shape=jax.ShapeDtypeStruct((NP, F), jnp.float32),
    )(scat2, gcat, dinv, b2)


def kernel(input, input_timestamp, edge_index, emb, W1, b1, W2, b2):
    del input_timestamp
    src = edge_index[0].astype(jnp.int32)
    dst = edge_index[1].astype(jnp.int32)
    npad = EPAD - E
    srcp = jnp.concatenate([src, jnp.zeros((npad,), jnp.int32)])
    padt = N + 64 + (jnp.arange(npad, dtype=jnp.int32) % (NP - N - 64))
    dstp = jnp.concatenate([dst, padt])
    src3 = srcp.reshape(EBLK, 128)
    dst3 = dstp.reshape(EBLK, 128)

    emb_p = jnp.zeros((NP, F), jnp.float32).at[:N].set(emb)

    degp = _deg_call(dstp).reshape(NC * NS, NP)
    dinv, x0cat = _prep_call(degp, emb_p)
    scat1 = _scatter_call(x0cat, src3, dst3)
    gcat = _mid_call(scat1, emb_p, dinv, W1, b1, W2)
    scat2 = _scatter_call(gcat, src3, dst3)
    table = _fin_call(scat2, gcat, dinv, b2)

    flat = input.reshape(-1).astype(jnp.int32)
    out = _gather_call(table, flat.reshape(NC * NS, _GBLK, 128))
    return out.reshape(input.shape[0], input.shape[1], F)
